# baseline (speedup 1.0000x reference)
import numpy as np
import jax
import jax.numpy as jnp
from jax.experimental import pallas as pl

_PC_X0 = 0.0
_PC_Y0 = -39.68
_VOX_X = 0.16
_VOX_Y = 0.16


def _loss_body(out_ref, m_ref, s_ref, gsel_ref, cls3_ref, wts_ref,
               o1_ref, o2_ref, o3_ref):
    _, C, Hc, W = out_ref.shape
    M = s_ref.shape[2]
    W9 = W // 9
    Rc = C * Hc

    X = out_ref[0].reshape(Rc, W)
    m_col = m_ref[0].reshape(Rc, 1)
    S = s_ref[0]
    gsel = gsel_ref[0]
    cls3 = cls3_ref[0]
    ones1r = jnp.ones((1, Rc), jnp.float32)

    Xm = m_col * X
    q_m = jnp.dot(ones1r, Xm * X, preferred_element_type=jnp.float32)
    r2 = jnp.dot(q_m, cls3, preferred_element_type=jnp.float32)

    XG = jnp.dot(X, gsel, preferred_element_type=jnp.float32)
    pg = jax.nn.sigmoid(XG[:, :W9])
    gmat = -jnp.clip(jnp.log(1.0 - pg), -100.0, None)
    row_g = jnp.sum(gmat, axis=1, keepdims=True)
    z2 = jnp.concatenate([row_g, m_col * row_g], axis=1)
    r1 = jnp.dot(ones1r, z2, preferred_element_type=jnp.float32)

    U3 = XG[:, W9:].reshape(C, Hc, M)
    ones1h = jnp.ones((1, Hc), jnp.float32)
    acc5 = jnp.zeros((5, M), jnp.float32)
    for c in range(C):
        v = jnp.dot(ones1h, S * U3[c], preferred_element_type=jnp.float32)
        acc5 = acc5 + wts_ref[0, :, c, :] * v
    r3 = jnp.dot(acc5, jnp.ones((M, 1), jnp.float32),
                 preferred_element_type=jnp.float32)

    s_id = pl.program_id(1)

    @pl.when(s_id == 0)
    def _():
        o1_ref[0] = r1
        o2_ref[0] = r2
        o3_ref[0] = r3

    @pl.when(s_id != 0)
    def _():
        o1_ref[0] = o1_ref[0] + r1
        o2_ref[0] = o2_ref[0] + r2
        o3_ref[0] = o3_ref[0] + r3


def kernel(outputs, targets, calibs, grid_size):
    B, C, H, W = outputs.shape
    M = targets.shape[1]
    W9 = W // 9
    R = C * H

    gs = grid_size.astype(jnp.float32)
    gr_y = H / gs[1]
    gr_x = W / gs[2]
    car = targets[:, :, 0] == 0.0
    xyz1 = jnp.concatenate(
        [targets[:, :, 11:14], jnp.ones((B, M, 1), jnp.float32)], axis=2)
    velo = jnp.einsum('bmk,bjk->bmj', xyz1, calibs)
    y_f = (velo[:, :, 1] - _PC_Y0) / _VOX_Y * gr_y
    x_f = (velo[:, :, 0] - _PC_X0) / _VOX_X * gr_x
    y_idx = jnp.clip(y_f.astype(jnp.int32), 0, H - 1)
    x_idx = jnp.clip(x_f.astype(jnp.int32), 0, W - 1)

    key = y_idx * W + x_idx
    mi = jnp.arange(M)
    eq = (key[:, :, None] == key[:, None, :]) & car[:, :, None] & car[:, None, :]
    later = eq & (mi[None, None, :] > mi[None, :, None])
    win = (car & ~jnp.any(later, axis=2)).astype(jnp.float32)

    rows = jnp.concatenate(
        [jnp.where(car, y_idx, H), jnp.where(car, x_idx, H)], axis=1)
    rm = jnp.any(jnp.arange(H)[None, :, None] == rows[:, None, :],
                 axis=2).astype(jnp.float32)
    m_flat = jnp.repeat(rm, 9, axis=1)
    m5 = m_flat.reshape(B, C, H, 1)
    nmt = 9.0 * jnp.sum(rm, axis=1)

    ys_s = jnp.where(car, y_idx, -1).astype(jnp.int32)
    xs_s = jnp.where(car, x_idx, -1).astype(jnp.int32)
    S = (jnp.arange(H)[None, :, None] == ys_s[:, None, :]).astype(jnp.float32)
    T = (jnp.arange(W)[None, :, None] == xs_s[:, None, :]).astype(jnp.float32)
    sel48 = (np.arange(W)[:, None] == 9 * np.arange(W9)[None, :]
             ).astype(np.float32)
    gsel = jnp.concatenate(
        [jnp.broadcast_to(jnp.asarray(sel48), (B, W, W9)), T], axis=2)

    wmod = np.arange(W) % 9
    cls3 = np.stack([((wmod >= 1) & (wmod <= 3)),
                     ((wmod >= 4) & (wmod <= 6)),
                     (wmod >= 7)], axis=1).astype(np.float32)
    cls3 = jnp.asarray(cls3)[None]

    cs = jnp.arange(C)
    idx_pm = jnp.clip((H * cs[None, :, None] + ys_s[:, None, :]) // 9,
                      0, H - 1)
    oneh = (idx_pm[:, :, :, None] == jnp.arange(H)[None, None, None, :]
            ).astype(jnp.float32)
    pm = jnp.einsum('bcmh,bh->bcm', oneh, rm)
    tv = jnp.stack([
        jnp.ones((B, M), jnp.float32),
        targets[:, :, 11], targets[:, :, 12], targets[:, :, 13],
        targets[:, :, 8], targets[:, :, 9], targets[:, :, 10],
        jnp.cos(targets[:, :, 14]), jnp.sin(targets[:, :, 14])], axis=1)
    xmod = xs_s % 9
    is_conf = (xmod == 0).astype(jnp.float32) * win
    is_coord = ((xmod >= 1) & (xmod <= 3)).astype(jnp.float32) * win
    is_shape = ((xmod >= 4) & (xmod <= 6)).astype(jnp.float32) * win
    is_angle = (xmod >= 7).astype(jnp.float32) * win
    icf = is_conf[:, None, :]
    wts = jnp.stack([
        -icf * pm * tv,
        -icf * (1.0 - pm) * tv,
        -2.0 * is_coord[:, None, :] * pm * tv,
        -2.0 * is_shape[:, None, :] * pm * tv,
        -2.0 * is_angle[:, None, :] * pm * tv], axis=1)
    const_coord = jnp.sum(is_coord[:, None, :] * pm * tv * tv, axis=(1, 2))
    const_shape = jnp.sum(is_shape[:, None, :] * pm * tv * tv, axis=(1, 2))
    const_angle = jnp.sum(is_angle[:, None, :] * pm * tv * tv, axis=(1, 2))

    NS = 2
    Hc = H // NS
    o1, o2, o3 = pl.pallas_call(
        _loss_body,
        grid=(B, NS),
        in_specs=[
            pl.BlockSpec((1, C, Hc, W), lambda b, s: (b, 0, s, 0)),
            pl.BlockSpec((1, C, Hc, 1), lambda b, s: (b, 0, s, 0)),
            pl.BlockSpec((1, Hc, M), lambda b, s: (b, s, 0)),
            pl.BlockSpec((1, W, W9 + M), lambda b, s: (b, 0, 0)),
            pl.BlockSpec((1, W, 3), lambda b, s: (0, 0, 0)),
            pl.BlockSpec((1, 5, C, M), lambda b, s: (b, 0, 0, 0)),
        ],
        out_specs=[
            pl.BlockSpec((1, 1, 2), lambda b, s: (b, 0, 0)),
            pl.BlockSpec((1, 1, 3), lambda b, s: (b, 0, 0)),
            pl.BlockSpec((1, 5, 1), lambda b, s: (b, 0, 0)),
        ],
        out_shape=[
            jax.ShapeDtypeStruct((B, 1, 2), jnp.float32),
            jax.ShapeDtypeStruct((B, 1, 3), jnp.float32),
            jax.ShapeDtypeStruct((B, 5, 1), jnp.float32),
        ],
    )(outputs, m5, S, gsel, cls3, wts)

    Gg = o1[:, 0, 0]
    Sgm = o1[:, 0, 1]
    mcs = o2[:, 0, :]
    corr = o3[:, :, 0]

    ln2 = -jnp.log(jnp.float32(0.5))
    n = C * H * W9
    N = B * n
    obj = jnp.sum(Sgm + (C * H - nmt) * W9 * ln2 + corr[:, 0]) / N
    noobj = 0.5 * jnp.sum((Gg - Sgm) + nmt * W9 * ln2 + corr[:, 1]) / N
    coord = 5.0 * jnp.sum(mcs[:, 0] + corr[:, 2] + const_coord) / (3 * N)
    shape_l = 5.0 * jnp.sum(mcs[:, 1] + corr[:, 3] + const_shape) / (3 * N)
    angle = 5.0 * jnp.sum(mcs[:, 2] + corr[:, 4] + const_angle) / (2 * N)
    return jnp.stack([obj, noobj, coord, shape_l, angle])


# speedup vs baseline: 1.2621x; 1.2621x over previous
import numpy as np
import jax
import jax.numpy as jnp
from jax.experimental import pallas as pl

_PC_X0 = 0.0
_PC_Y0 = -39.68
_VOX_X = 0.16
_VOX_Y = 0.16


def _loss_body(out_ref, m2t_ref, s_ref, gsel_ref, cls3_ref, wts_ref,
               o1_ref, o2_ref, o3_ref):
    _, C, H, W = out_ref.shape
    M = s_ref.shape[2]
    W9 = W // 9
    R = C * H

    X = out_ref[0].reshape(R, W)
    m2t = m2t_ref[0]
    S = s_ref[0]
    gsel = gsel_ref[0]
    cls3 = cls3_ref[0]

    SQ = X * X
    Q2 = jnp.dot(m2t, SQ, preferred_element_type=jnp.float32)
    o2_ref[0] = jnp.dot(Q2[1:2, :], cls3,
                        preferred_element_type=jnp.float32)

    XG = jnp.dot(X, gsel, preferred_element_type=jnp.float32)
    pg = jax.nn.sigmoid(XG[:, :W9])
    gmat = -jnp.clip(jnp.log(1.0 - pg), -100.0, None)
    row_g = jnp.sum(gmat, axis=1, keepdims=True)
    o1_ref[0] = jnp.dot(m2t, row_g, preferred_element_type=jnp.float32)

    U3 = XG[:, W9:].reshape(C, H, M)
    ones1h = jnp.ones((1, H), jnp.float32)
    acc5 = jnp.zeros((5, M), jnp.float32)
    for c in range(C):
        v = jnp.dot(ones1h, S * U3[c], preferred_element_type=jnp.float32)
        acc5 = acc5 + wts_ref[0, :, c, :] * v
    o3_ref[0] = jnp.dot(acc5, jnp.ones((M, 1), jnp.float32),
                        preferred_element_type=jnp.float32)


def kernel(outputs, targets, calibs, grid_size):
    B, C, H, W = outputs.shape
    M = targets.shape[1]
    W9 = W // 9
    R = C * H

    gs = grid_size.astype(jnp.float32)
    gr_y = H / gs[1]
    gr_x = W / gs[2]
    car = targets[:, :, 0] == 0.0
    xyz1 = jnp.concatenate(
        [targets[:, :, 11:14], jnp.ones((B, M, 1), jnp.float32)], axis=2)
    velo = jnp.einsum('bmk,bjk->bmj', xyz1, calibs)
    y_f = (velo[:, :, 1] - _PC_Y0) / _VOX_Y * gr_y
    x_f = (velo[:, :, 0] - _PC_X0) / _VOX_X * gr_x
    y_idx = jnp.clip(y_f.astype(jnp.int32), 0, H - 1)
    x_idx = jnp.clip(x_f.astype(jnp.int32), 0, W - 1)

    key = y_idx * W + x_idx
    mi = jnp.arange(M)
    eq = (key[:, :, None] == key[:, None, :]) & car[:, :, None] & car[:, None, :]
    later = eq & (mi[None, None, :] > mi[None, :, None])
    win = (car & ~jnp.any(later, axis=2)).astype(jnp.float32)

    rows = jnp.concatenate(
        [jnp.where(car, y_idx, H), jnp.where(car, x_idx, H)], axis=1)
    rm = jnp.any(jnp.arange(H)[None, :, None] == rows[:, None, :],
                 axis=2).astype(jnp.float32)
    m_flat = jnp.repeat(rm, 9, axis=1)
    m2t = jnp.stack([jnp.ones((B, R), jnp.float32), m_flat], axis=1)
    nmt = 9.0 * jnp.sum(rm, axis=1)

    ys_s = jnp.where(car, y_idx, -1).astype(jnp.int32)
    xs_s = jnp.where(car, x_idx, -1).astype(jnp.int32)
    S = (jnp.arange(H)[None, :, None] == ys_s[:, None, :]).astype(jnp.float32)
    T = (jnp.arange(W)[None, :, None] == xs_s[:, None, :]).astype(jnp.float32)
    sel48 = (np.arange(W)[:, None] == 9 * np.arange(W9)[None, :]
             ).astype(np.float32)
    gsel = jnp.concatenate(
        [jnp.broadcast_to(jnp.asarray(sel48), (B, W, W9)), T], axis=2)

    wmod = np.arange(W) % 9
    cls3 = np.stack([((wmod >= 1) & (wmod <= 3)),
                     ((wmod >= 4) & (wmod <= 6)),
                     (wmod >= 7)], axis=1).astype(np.float32)
    cls3 = jnp.asarray(cls3)[None]

    cs = jnp.arange(C)
    idx_pm = jnp.clip((H * cs[None, :, None] + ys_s[:, None, :]) // 9,
                      0, H - 1)
    oneh = (idx_pm[:, :, :, None] == jnp.arange(H)[None, None, None, :]
            ).astype(jnp.float32)
    pm = jnp.einsum('bcmh,bh->bcm', oneh, rm)
    tv = jnp.stack([
        jnp.ones((B, M), jnp.float32),
        targets[:, :, 11], targets[:, :, 12], targets[:, :, 13],
        targets[:, :, 8], targets[:, :, 9], targets[:, :, 10],
        jnp.cos(targets[:, :, 14]), jnp.sin(targets[:, :, 14])], axis=1)
    xmod = xs_s % 9
    is_conf = (xmod == 0).astype(jnp.float32) * win
    is_coord = ((xmod >= 1) & (xmod <= 3)).astype(jnp.float32) * win
    is_shape = ((xmod >= 4) & (xmod <= 6)).astype(jnp.float32) * win
    is_angle = (xmod >= 7).astype(jnp.float32) * win
    icf = is_conf[:, None, :]
    wts = jnp.stack([
        -icf * pm * tv,
        -icf * (1.0 - pm) * tv,
        -2.0 * is_coord[:, None, :] * pm * tv,
        -2.0 * is_shape[:, None, :] * pm * tv,
        -2.0 * is_angle[:, None, :] * pm * tv], axis=1)
    const_coord = jnp.sum(is_coord[:, None, :] * pm * tv * tv, axis=(1, 2))
    const_shape = jnp.sum(is_shape[:, None, :] * pm * tv * tv, axis=(1, 2))
    const_angle = jnp.sum(is_angle[:, None, :] * pm * tv * tv, axis=(1, 2))

    o1, o2, o3 = pl.pallas_call(
        _loss_body,
        grid=(B,),
        in_specs=[
            pl.BlockSpec((1, C, H, W), lambda b: (b, 0, 0, 0)),
            pl.BlockSpec((1, 2, R), lambda b: (b, 0, 0)),
            pl.BlockSpec((1, H, M), lambda b: (b, 0, 0)),
            pl.BlockSpec((1, W, W9 + M), lambda b: (b, 0, 0)),
            pl.BlockSpec((1, W, 3), lambda b: (0, 0, 0)),
            pl.BlockSpec((1, 5, C, M), lambda b: (b, 0, 0, 0)),
        ],
        out_specs=[
            pl.BlockSpec((1, 2, 1), lambda b: (b, 0, 0)),
            pl.BlockSpec((1, 1, 3), lambda b: (b, 0, 0)),
            pl.BlockSpec((1, 5, 1), lambda b: (b, 0, 0)),
        ],
        out_shape=[
            jax.ShapeDtypeStruct((B, 2, 1), jnp.float32),
            jax.ShapeDtypeStruct((B, 1, 3), jnp.float32),
            jax.ShapeDtypeStruct((B, 5, 1), jnp.float32),
        ],
    )(outputs, m2t, S, gsel, cls3, wts)

    Gg = o1[:, 0, 0]
    Sgm = o1[:, 1, 0]
    mcs = o2[:, 0, :]
    corr = o3[:, :, 0]

    ln2 = -jnp.log(jnp.float32(0.5))
    n = C * H * W9
    N = B * n
    obj = jnp.sum(Sgm + (C * H - nmt) * W9 * ln2 + corr[:, 0]) / N
    noobj = 0.5 * jnp.sum((Gg - Sgm) + nmt * W9 * ln2 + corr[:, 1]) / N
    coord = 5.0 * jnp.sum(mcs[:, 0] + corr[:, 2] + const_coord) / (3 * N)
    shape_l = 5.0 * jnp.sum(mcs[:, 1] + corr[:, 3] + const_shape) / (3 * N)
    angle = 5.0 * jnp.sum(mcs[:, 2] + corr[:, 4] + const_angle) / (2 * N)
    return jnp.stack([obj, noobj, coord, shape_l, angle])
